# baseline (speedup 1.0000x reference)
import jax
import jax.numpy as jnp
from jax.experimental import pallas as pl
from jax.experimental.pallas import tpu as pltpu

_K = 2
_CCB = 32
_SUB = 8


def _pool_body(x_ref, o_ref):
    ccb = x_ref.shape[1]
    H = x_ref.shape[2]
    W = x_ref.shape[3]
    C = o_ref.shape[1] // 3
    h, w = H // _K, W // _K
    zero = jnp.float32(0.0)
    one = jnp.float32(1.0)
    inv_h = jnp.float32(1.0 / H)
    inv_w = jnp.float32(1.0 / W)
    sub = min(_SUB, ccb)
    base = pl.program_id(1) * ccb

    def _chunk(i, _):
        c0 = i * sub
        cs = pl.ds(c0, sub)
        a = x_ref[0, cs, pl.Slice(0, h, _K), :]
        b = x_ref[0, cs, pl.Slice(1, h, _K), :]

        sa = pltpu.roll(a, W - 1, 2)
        sb = pltpu.roll(b, W - 1, 2)
        gea = a >= sa
        geb = b >= sb
        ma = jnp.where(gea, a, sa)
        mb = jnp.where(geb, b, sb)

        gev = ma >= mb
        vals_f = jnp.where(gev, ma, mb)
        dh_f = jnp.where(gev, zero, one)
        dw_f = jnp.where(gev,
                         jnp.where(gea, zero, one),
                         jnp.where(geb, zero, one))
        code_f = dh_f + dh_f + dw_f

        idx = jax.lax.broadcasted_iota(jnp.int32, (sub, h, w), 2) * 2
        vals = jnp.take_along_axis(vals_f, idx, axis=2)
        code = jnp.take_along_axis(code_f, idx, axis=2)

        dh = jnp.floor(code * jnp.float32(0.5))
        dw = code - (dh + dh)
        ww = jax.lax.broadcasted_iota(jnp.int32, (sub, h, w), 2).astype(jnp.float32)
        hh = jax.lax.broadcasted_iota(jnp.int32, (sub, h, w), 1).astype(jnp.float32)
        pos_h = (hh * _K + dh) * inv_h
        pos_w = (ww * _K + dw) * inv_w

        o_ref[0, pl.ds(base + c0, sub)] = vals
        o_ref[0, pl.ds(C + base + c0, sub)] = pos_h
        o_ref[0, pl.ds(2 * C + base + c0, sub)] = pos_w
        return 0

    jax.lax.fori_loop(0, ccb // sub, _chunk, 0)


def kernel(x):
    B, C, H, W = x.shape
    h, w = H // _K, W // _K
    ccb = min(_CCB, C)
    return pl.pallas_call(
        _pool_body,
        out_shape=jax.ShapeDtypeStruct((B, 3 * C, h, w), x.dtype),
        grid=(B, C // ccb),
        in_specs=[pl.BlockSpec((1, ccb, H, W), lambda b, c: (b, c, 0, 0))],
        out_specs=pl.BlockSpec((1, 3 * C, h, w), lambda b, c: (b, 0, 0, 0)),
        compiler_params=pltpu.CompilerParams(
            dimension_semantics=("parallel", "arbitrary"),
            vmem_limit_bytes=48 * 1024 * 1024,
        ),
        name="pos_aware_pool",
    )(x)


# speedup vs baseline: 1.1689x; 1.1689x over previous
import jax
import jax.numpy as jnp
from jax.experimental import pallas as pl
from jax.experimental.pallas import tpu as pltpu

_K = 2
_CCB = 128
_SUB = 8


def _pool_body(x_ref, o_ref):
    ccb = x_ref.shape[1]
    H = x_ref.shape[2]
    W = x_ref.shape[3]
    C = o_ref.shape[1] // 3
    h, w = H // _K, W // _K
    zero = jnp.float32(0.0)
    one = jnp.float32(1.0)
    inv_h = jnp.float32(1.0 / H)
    inv_w = jnp.float32(1.0 / W)
    sub = min(_SUB, ccb)
    base = pl.program_id(1) * ccb

    def _chunk(i, _):
        c0 = i * sub
        cs = pl.ds(c0, sub)
        a = x_ref[0, cs, pl.Slice(0, h, _K), :]
        b = x_ref[0, cs, pl.Slice(1, h, _K), :]

        sa = pltpu.roll(a, W - 1, 2)
        sb = pltpu.roll(b, W - 1, 2)
        gea = a >= sa
        geb = b >= sb
        ma = jnp.where(gea, a, sa)
        mb = jnp.where(geb, b, sb)

        gev = ma >= mb
        vals_f = jnp.where(gev, ma, mb)
        dh_f = jnp.where(gev, zero, one)
        dw_f = jnp.where(gev,
                         jnp.where(gea, zero, one),
                         jnp.where(geb, zero, one))
        code_f = dh_f + dh_f + dw_f

        idx = jax.lax.broadcasted_iota(jnp.int32, (sub, h, w), 2) * 2
        vals = jnp.take_along_axis(vals_f, idx, axis=2)
        code = jnp.take_along_axis(code_f, idx, axis=2)

        dh = jnp.floor(code * jnp.float32(0.5))
        dw = code - (dh + dh)
        ww = jax.lax.broadcasted_iota(jnp.int32, (sub, h, w), 2).astype(jnp.float32)
        hh = jax.lax.broadcasted_iota(jnp.int32, (sub, h, w), 1).astype(jnp.float32)
        pos_h = (hh * _K + dh) * inv_h
        pos_w = (ww * _K + dw) * inv_w

        o_ref[0, pl.ds(base + c0, sub)] = vals
        o_ref[0, pl.ds(C + base + c0, sub)] = pos_h
        o_ref[0, pl.ds(2 * C + base + c0, sub)] = pos_w
        return 0

    jax.lax.fori_loop(0, ccb // sub, _chunk, 0)


def kernel(x):
    B, C, H, W = x.shape
    h, w = H // _K, W // _K
    ccb = min(_CCB, C)
    return pl.pallas_call(
        _pool_body,
        out_shape=jax.ShapeDtypeStruct((B, 3 * C, h, w), x.dtype),
        grid=(B, C // ccb),
        in_specs=[pl.BlockSpec((1, ccb, H, W), lambda b, c: (b, c, 0, 0))],
        out_specs=pl.BlockSpec((1, 3 * C, h, w), lambda b, c: (b, 0, 0, 0)),
        compiler_params=pltpu.CompilerParams(
            dimension_semantics=("parallel", "arbitrary"),
            vmem_limit_bytes=48 * 1024 * 1024,
        ),
        name="pos_aware_pool",
    )(x)


# speedup vs baseline: 1.3658x; 1.1685x over previous
import jax
import jax.numpy as jnp
from jax.experimental import pallas as pl
from jax.experimental.pallas import tpu as pltpu

_K = 2
_CCB = 32
_SUB = 8


def _pool_body(x_ref, o_ref):
    ccb = x_ref.shape[1]
    H = x_ref.shape[2]
    W = x_ref.shape[3]
    h, w = H // _K, W // _K
    zero = jnp.float32(0.0)
    one = jnp.float32(1.0)
    inv_h = jnp.float32(1.0 / H)
    inv_w = jnp.float32(1.0 / W)
    sub = min(_SUB, ccb)

    def _chunk(i, _):
        c0 = i * sub
        cs = pl.ds(c0, sub)
        a = x_ref[0, cs, pl.Slice(0, h, _K), :]
        b = x_ref[0, cs, pl.Slice(1, h, _K), :]

        sa = pltpu.roll(a, W - 1, 2)
        sb = pltpu.roll(b, W - 1, 2)
        gea = a >= sa
        geb = b >= sb
        ma = jnp.where(gea, a, sa)
        mb = jnp.where(geb, b, sb)

        gev = ma >= mb
        vals_f = jnp.where(gev, ma, mb)
        dh_f = jnp.where(gev, zero, one)
        dw_f = jnp.where(gev,
                         jnp.where(gea, zero, one),
                         jnp.where(geb, zero, one))
        code_f = dh_f + dh_f + dw_f

        idx = jax.lax.broadcasted_iota(jnp.int32, (sub, h, w), 2) * 2
        vals = jnp.take_along_axis(vals_f, idx, axis=2)
        code = jnp.take_along_axis(code_f, idx, axis=2)

        dh = jnp.floor(code * jnp.float32(0.5))
        dw = code - (dh + dh)
        ww = jax.lax.broadcasted_iota(jnp.int32, (sub, h, w), 2).astype(jnp.float32)
        hh = jax.lax.broadcasted_iota(jnp.int32, (sub, h, w), 1).astype(jnp.float32)
        pos_h = (hh * _K + dh) * inv_h
        pos_w = (ww * _K + dw) * inv_w

        o_ref[0, 0, cs] = vals
        o_ref[0, 1, cs] = pos_h
        o_ref[0, 2, cs] = pos_w
        return 0

    jax.lax.fori_loop(0, ccb // sub, _chunk, 0)


def kernel(x):
    B, C, H, W = x.shape
    h, w = H // _K, W // _K
    ccb = min(_CCB, C)
    out = pl.pallas_call(
        _pool_body,
        out_shape=jax.ShapeDtypeStruct((B, 3, C, h, w), x.dtype),
        grid=(B, C // ccb),
        in_specs=[pl.BlockSpec((1, ccb, H, W), lambda b, c: (b, c, 0, 0))],
        out_specs=pl.BlockSpec((1, 3, ccb, h, w), lambda b, c: (b, 0, c, 0, 0)),
        compiler_params=pltpu.CompilerParams(
            dimension_semantics=("parallel", "arbitrary"),
            vmem_limit_bytes=40 * 1024 * 1024,
        ),
        name="pos_aware_pool",
    )(x)
    return out.reshape(B, 3 * C, h, w)
